# baseline (speedup 1.0000x reference)
import functools

import jax
import jax.numpy as jnp
from jax import lax
from jax.experimental import pallas as pl
from jax.experimental.pallas import tpu as pltpu
from jax.experimental.pallas import tpu_sc as plsc

_NUM_CLASSES = 12
_FEATS = 16

_AGENT_SET = (1, 4, 5, 6, 7, 8, 9, 11)
_BOX_SET = (2, 5, 8, 9, 10, 11)
_TARGET_SET = (3, 6, 7, 8, 9, 10)
_CARRY_SET = (4, 7, 9, 11)

_SEM_COLS = [[1.0 if v in s else 0.0 for s in
              (_AGENT_SET, _BOX_SET, _TARGET_SET, _CARRY_SET)]
             for v in range(_NUM_CLASSES)]

_NC = 2
_NS = 16
_NW = _NC * _NS

_CHUNK = 2048
_GROUP = 128


def _sc_kernel(n):
    mesh = plsc.VectorSubcoreMesh(core_axis_name="c", subcore_axis_name="s")
    b_per_w = n // _NW

    @functools.partial(
        pl.kernel, mesh=mesh,
        out_type=jax.ShapeDtypeStruct((n, _FEATS), jnp.float32),
        scratch_types=[
            pltpu.VMEM((_CHUNK,), jnp.int32),
            pltpu.VMEM((_CHUNK, _FEATS), jnp.float32),
            pltpu.SemaphoreType.DMA,
        ],
        compiler_params=pltpu.CompilerParams(use_tc_tiling_on_sc=False),
    )
    def k(lut_hbm, idx_hbm, out_hbm, idx_v, rows_v, sem):
        wid = lax.axis_index("s") * _NC + lax.axis_index("c")
        base = wid * b_per_w

        def body(i, carry):
            o = base + i * _CHUNK
            pltpu.sync_copy(idx_hbm.at[pl.ds(o, _CHUNK)], idx_v)
            copies = []
            for j in range(_CHUNK // _GROUP):
                copies.append(pltpu.async_copy(
                    lut_hbm.at[idx_v.at[pl.ds(j * _GROUP, _GROUP)]],
                    rows_v.at[pl.ds(j * _GROUP, _GROUP)], sem))
            for cp in copies:
                cp.wait()
            pltpu.sync_copy(rows_v, out_hbm.at[pl.ds(o, _CHUNK)])
            return carry

        lax.fori_loop(0, b_per_w // _CHUNK, body, 0)

    return k


def kernel(x, table):
    b, c = x.shape
    n = b * c
    lut = jnp.concatenate(
        [table.astype(jnp.float32),
         jnp.asarray(_SEM_COLS, dtype=jnp.float32)], axis=1)
    out = _sc_kernel(n)(lut, x.reshape(n))
    return out.reshape(b, c, _FEATS)
